# Initial kernel scaffold; baseline (speedup 1.0000x reference)
import jax
import jax.numpy as jnp
from jax.experimental import pallas as pl


def kernel(normalized_adjacency_matrix, features, weight_matrix):
    raise NotImplementedError("write your pallas kernel here")


# speedup vs baseline: 1.0824x; 1.0824x over previous
import functools

import jax
import jax.numpy as jnp
from jax.experimental import pallas as pl


def _base_kernel(f_ref, w_ref, o_ref):
    b = jnp.dot(f_ref[...], w_ref[...], preferred_element_type=jnp.float32)
    o_ref[...] = jnp.maximum(b, 0.0).astype(jnp.bfloat16)


def _prop_kernel(a_ref, x_ref, o_ref, *, out_dtype):
    acc = jnp.dot(
        a_ref[...].astype(jnp.bfloat16),
        x_ref[...],
        preferred_element_type=jnp.float32,
    )
    o_ref[...] = acc.astype(out_dtype)


def _propagate(a, x, out_dtype, bm):
    n = a.shape[0]
    grid = (n // bm,)
    return pl.pallas_call(
        functools.partial(_prop_kernel, out_dtype=out_dtype),
        grid=grid,
        in_specs=[
            pl.BlockSpec((bm, n), lambda i: (i, 0)),
            pl.BlockSpec((n, x.shape[1]), lambda i: (0, 0)),
        ],
        out_specs=pl.BlockSpec((bm, x.shape[1]), lambda i: (i, 0)),
        out_shape=jax.ShapeDtypeStruct((n, x.shape[1]), out_dtype),
    )(a, x)


def kernel(normalized_adjacency_matrix, features, weight_matrix):
    a = normalized_adjacency_matrix
    n, c_in = features.shape
    c_out = weight_matrix.shape[1]

    base = pl.pallas_call(
        _base_kernel,
        out_shape=jax.ShapeDtypeStruct((n, c_out), jnp.bfloat16),
    )(features, weight_matrix)

    y1 = _propagate(a, base, jnp.bfloat16, bm=400)
    y2 = _propagate(a, y1, jnp.float32, bm=400)
    return y2
